# baseline (speedup 1.0000x reference)
import functools

import numpy as np
import jax
import jax.numpy as jnp
from jax import lax
from jax.experimental import pallas as pl
from jax.experimental.pallas import tpu as pltpu
from jax.experimental.pallas import tpu_sc as plsc

_N_ROWS = 16384
_SLICE = 4096
_SL, _LN = 26, 128

_FLAT_IN = _N_ROWS * _SL
_FLAT_OUT = _SLICE * _SL

_NC, _NS = 2, 16
_NW = _NC * _NS
_R_PER_W = _FLAT_OUT // _NW
_CHUNK = 128
_NCHUNK = _R_PER_W // _CHUNK
_PAIRS = _NCHUNK // 2

_index_cache = None


def _perm_index() -> np.ndarray:
    global _index_cache
    if _index_cache is None:
        with jax.ensure_compile_time_eval():
            perm_key = jax.random.fold_in(jax.random.key(0), 1)
            perm = jax.random.permutation(perm_key, _N_ROWS)[:_SLICE]
        _index_cache = np.asarray(perm)
    return _index_cache


def _sc_gather(table, idx):
    mesh = plsc.VectorSubcoreMesh(core_axis_name="c", subcore_axis_name="s")

    @functools.partial(
        pl.kernel,
        mesh=mesh,
        out_type=jax.ShapeDtypeStruct((_FLAT_OUT, _LN), jnp.float32),
        scratch_types=[
            pltpu.VMEM((_NCHUNK, _CHUNK), jnp.int32),
            pltpu.VMEM((_CHUNK, _LN), jnp.float32),
            pltpu.VMEM((_CHUNK, _LN), jnp.float32),
            pltpu.SemaphoreType.DMA,
            pltpu.SemaphoreType.DMA,
            pltpu.SemaphoreType.DMA,
            pltpu.SemaphoreType.DMA,
        ],
    )
    def k(table_hbm, idx_hbm, out_hbm, idx_v,
          bufa, bufb, ga, gb, oa, ob):
        wid = lax.axis_index("s") * _NC + lax.axis_index("c")
        base = wid * _R_PER_W
        pltpu.sync_copy(idx_hbm.at[wid], idx_v)

        def gather(c, buf, sem):
            return pltpu.async_copy(table_hbm.at[idx_v.at[c]], buf, sem)

        def put(c, buf, sem):
            return pltpu.async_copy(
                buf, out_hbm.at[pl.ds(base + c * _CHUNK, _CHUNK)], sem)

        def drain(buf, sem):
            pltpu.make_async_copy(
                buf, out_hbm.at[pl.ds(base, _CHUNK)], sem).wait()

        def step(g, carry):
            c0 = 2 * g

            @pl.when(g > 0)
            def _():
                drain(bufa, oa)

            g0 = gather(c0, bufa, ga)

            @pl.when(g > 0)
            def _():
                drain(bufb, ob)

            g1 = gather(c0 + 1, bufb, gb)
            g0.wait()
            put(c0, bufa, oa)
            g1.wait()
            put(c0 + 1, bufb, ob)
            return carry

        lax.fori_loop(0, _PAIRS, step, 0)
        drain(bufa, oa)
        drain(bufb, ob)

    return k(table, idx)


def kernel(x):
    idx_np = _perm_index()
    flat_idx = (np.arange(_SL, dtype=np.int64)[:, None] * _N_ROWS
                + idx_np[None, :].astype(np.int64)).reshape(-1)
    idx_dev = jnp.asarray(flat_idx, dtype=jnp.int32).reshape(
        _NW, _NCHUNK, _CHUNK)
    table = jnp.transpose(x, (1, 0, 2)).reshape(_FLAT_IN, _LN)
    out_flat = _sc_gather(table, idx_dev)
    out = jnp.transpose(out_flat.reshape(_SL, _SLICE, _LN), (1, 0, 2))
    return (out, jnp.asarray(idx_np))


# speedup vs baseline: 1.0559x; 1.0559x over previous
import functools

import numpy as np
import jax
import jax.numpy as jnp
from jax import lax
from jax.experimental import pallas as pl
from jax.experimental.pallas import tpu as pltpu
from jax.experimental.pallas import tpu_sc as plsc

_N_ROWS = 16384
_SLICE = 4096
_SL, _LN = 26, 128

_FLAT_IN = _N_ROWS * _SL
_FLAT_OUT = _SLICE * _SL

_NC, _NS = 2, 16
_NW = _NC * _NS
_R_PER_W = _FLAT_OUT // _NW
_CHUNK = 104
_NCHUNK = _R_PER_W // _CHUNK
_NBUF = 8
_STEPS = _NCHUNK // _NBUF

_index_cache = None


def _perm_index() -> np.ndarray:
    global _index_cache
    if _index_cache is None:
        with jax.ensure_compile_time_eval():
            perm_key = jax.random.fold_in(jax.random.key(0), 1)
            perm = jax.random.permutation(perm_key, _N_ROWS)[:_SLICE]
        _index_cache = np.asarray(perm)
    return _index_cache


def _sc_gather(table, idx):
    mesh = plsc.VectorSubcoreMesh(core_axis_name="c", subcore_axis_name="s")

    @functools.partial(
        pl.kernel,
        mesh=mesh,
        out_type=jax.ShapeDtypeStruct((_FLAT_OUT, _LN), jnp.float32),
        scratch_types=(
            [pltpu.VMEM((_NCHUNK, _CHUNK), jnp.int32)]
            + [pltpu.VMEM((_CHUNK, _LN), jnp.float32)] * _NBUF
            + [pltpu.SemaphoreType.DMA] * (2 * _NBUF)
        ),
    )
    def k(table_hbm, idx_hbm, out_hbm, idx_v, *rest):
        bufs = rest[:_NBUF]
        gsems = rest[_NBUF:2 * _NBUF]
        osems = rest[2 * _NBUF:]
        wid = lax.axis_index("s") * _NC + lax.axis_index("c")
        base = wid * _R_PER_W
        pltpu.sync_copy(idx_hbm.at[wid], idx_v)

        def gather(c, buf, sem):
            return pltpu.async_copy(table_hbm.at[idx_v.at[c]], buf, sem)

        def put(c, buf, sem):
            return pltpu.async_copy(
                buf, out_hbm.at[pl.ds(base + c * _CHUNK, _CHUNK)], sem)

        def drain(buf, sem):
            pltpu.make_async_copy(
                buf, out_hbm.at[pl.ds(base, _CHUNK)], sem).wait()

        def step(g, carry):
            c0 = _NBUF * g
            gs = []
            for b in range(_NBUF):
                @pl.when(g > 0)
                def _(b=b):
                    drain(bufs[b], osems[b])
                gs.append(gather(c0 + b, bufs[b], gsems[b]))
            for b in range(_NBUF):
                gs[b].wait()
                put(c0 + b, bufs[b], osems[b])
            return carry

        lax.fori_loop(0, _STEPS, step, 0)
        for b in range(_NBUF):
            drain(bufs[b], osems[b])

    return k(table, idx)


def kernel(x):
    idx_np = _perm_index()
    flat_idx = (np.arange(_SL, dtype=np.int64)[:, None] * _N_ROWS
                + idx_np[None, :].astype(np.int64)).reshape(-1)
    idx_dev = jnp.asarray(flat_idx, dtype=jnp.int32).reshape(
        _NW, _NCHUNK, _CHUNK)
    table = jnp.transpose(x, (1, 0, 2)).reshape(_FLAT_IN, _LN)
    out_flat = _sc_gather(table, idx_dev)
    out = jnp.transpose(out_flat.reshape(_SL, _SLICE, _LN), (1, 0, 2))
    return (out, jnp.asarray(idx_np))


# speedup vs baseline: 1.0808x; 1.0236x over previous
import functools

import numpy as np
import jax
import jax.numpy as jnp
from jax import lax
from jax.experimental import pallas as pl
from jax.experimental.pallas import tpu as pltpu
from jax.experimental.pallas import tpu_sc as plsc

_N_ROWS = 16384
_SLICE = 4096
_SL, _LN = 26, 128

_FLAT_IN = _N_ROWS * _SL
_FLAT_OUT = _SLICE * _SL

_NC, _NS = 2, 16
_NW = _NC * _NS
_R_PER_W = _FLAT_OUT // _NW
_CHUNK = 104
_NCHUNK = _R_PER_W // _CHUNK
_QUADS = _NCHUNK // 4

_index_cache = None


def _perm_index() -> np.ndarray:
    global _index_cache
    if _index_cache is None:
        with jax.ensure_compile_time_eval():
            perm_key = jax.random.fold_in(jax.random.key(0), 1)
            perm = jax.random.permutation(perm_key, _N_ROWS)[:_SLICE]
        _index_cache = np.asarray(perm)
    return _index_cache


def _sc_gather(table, idx):
    mesh = plsc.VectorSubcoreMesh(core_axis_name="c", subcore_axis_name="s")

    @functools.partial(
        pl.kernel,
        mesh=mesh,
        out_type=jax.ShapeDtypeStruct((_FLAT_OUT, _LN), jnp.float32),
        scratch_types=[
            pltpu.VMEM((_NCHUNK, _CHUNK), jnp.int32),
            pltpu.VMEM((_CHUNK, _LN), jnp.float32),
            pltpu.VMEM((_CHUNK, _LN), jnp.float32),
            pltpu.VMEM((_CHUNK, _LN), jnp.float32),
            pltpu.VMEM((_CHUNK, _LN), jnp.float32),
            pltpu.SemaphoreType.DMA,
            pltpu.SemaphoreType.DMA,
            pltpu.SemaphoreType.DMA,
            pltpu.SemaphoreType.DMA,
            pltpu.SemaphoreType.DMA,
            pltpu.SemaphoreType.DMA,
            pltpu.SemaphoreType.DMA,
            pltpu.SemaphoreType.DMA,
        ],
    )
    def k(table_hbm, idx_hbm, out_hbm, idx_v,
          bufa0, bufa1, bufb0, bufb1,
          ga0, ga1, gb0, gb1, oa0, oa1, ob0, ob1):
        wid = lax.axis_index("s") * _NC + lax.axis_index("c")
        base = wid * _R_PER_W
        pltpu.sync_copy(idx_hbm.at[wid], idx_v)

        def gather(c, buf, sem):
            return pltpu.async_copy(table_hbm.at[idx_v.at[c]], buf, sem)

        def put(c, buf, sem):
            return pltpu.async_copy(
                buf, out_hbm.at[pl.ds(base + c * _CHUNK, _CHUNK)], sem)

        def drain(buf, sem):
            pltpu.make_async_copy(
                buf, out_hbm.at[pl.ds(base, _CHUNK)], sem).wait()

        def step(g, carry):
            c0 = 4 * g

            @pl.when(g > 0)
            def _():
                drain(bufa0, oa0)
                drain(bufa1, oa1)

            g0 = gather(c0, bufa0, ga0)
            g1 = gather(c0 + 1, bufa1, ga1)

            @pl.when(g > 0)
            def _():
                drain(bufb0, ob0)
                drain(bufb1, ob1)

            g0.wait()
            put(c0, bufa0, oa0)
            g2 = gather(c0 + 2, bufb0, gb0)
            g1.wait()
            put(c0 + 1, bufa1, oa1)
            g3 = gather(c0 + 3, bufb1, gb1)
            g2.wait()
            put(c0 + 2, bufb0, ob0)
            g3.wait()
            put(c0 + 3, bufb1, ob1)
            return carry

        lax.fori_loop(0, _QUADS, step, 0)
        drain(bufa0, oa0)
        drain(bufa1, oa1)
        drain(bufb0, ob0)
        drain(bufb1, ob1)

    return k(table, idx)


def kernel(x):
    idx_np = _perm_index()
    flat_idx = (np.arange(_SL, dtype=np.int64)[:, None] * _N_ROWS
                + idx_np[None, :].astype(np.int64)).reshape(-1)
    idx_dev = jnp.asarray(flat_idx, dtype=jnp.int32).reshape(
        _NW, _NCHUNK, _CHUNK)
    table = jnp.transpose(x, (1, 0, 2)).reshape(_FLAT_IN, _LN)
    out_flat = _sc_gather(table, idx_dev)
    out = jnp.transpose(out_flat.reshape(_SL, _SLICE, _LN), (1, 0, 2))
    return (out, jnp.asarray(idx_np))
